# Initial kernel scaffold; baseline (speedup 1.0000x reference)
import jax
import jax.numpy as jnp
from jax.experimental import pallas as pl


def kernel(x, edge_index, W1, b1, W2, b2):
    raise NotImplementedError("write your pallas kernel here")


# speedup vs baseline: 15.2307x; 15.2307x over previous
import jax
import jax.numpy as jnp
from jax import lax
from jax.experimental import pallas as pl
from jax.experimental.pallas import tpu as pltpu
from jax.experimental.pallas import tpu_sc as plsc

NC, NS = 2, 16
NW = NC * NS
CH = 128


def _sc_mesh():
    return plsc.VectorSubcoreMesh(core_axis_name="c", subcore_axis_name="s")


def _deg_call(dst, n_nodes):
    E = dst.shape[0]
    e_sc, e_w = E // NC, E // NW
    n_full, tail = divmod(e_w, CH)
    rpt = ((n_nodes + NS - 1) // NS + 127) // 128 * 128
    n_pad = NS * rpt
    zeros = jnp.zeros((rpt,), jnp.float32)
    ones = jnp.ones((CH,), jnp.float32)

    def body(dst_hbm, zeros_hbm, ones_hbm, out_hbm,
             acc, stage, dst_v, ones_v, dst_t, ones_t):
        cid = lax.axis_index("c")
        sid = lax.axis_index("s")
        my = pl.ds(sid * rpt, rpt)
        pltpu.sync_copy(zeros_hbm, stage)
        pltpu.sync_copy(stage, acc.at[my])
        pltpu.sync_copy(ones_hbm, ones_v)
        if tail:
            pltpu.sync_copy(ones_hbm.at[pl.ds(0, tail)], ones_t)
        plsc.subcore_barrier()
        base = cid * e_sc + sid * e_w

        def chunk(i):
            off = pl.multiple_of(base + i * CH, 8)
            pltpu.sync_copy(dst_hbm.at[pl.ds(off, CH)], dst_v)
            pltpu.sync_copy(ones_v, acc.at[dst_v], add=True)

        lax.fori_loop(0, n_full, lambda i, c: (chunk(i), c)[1], 0)
        if tail:
            off = pl.multiple_of(base + n_full * CH, 8)
            pltpu.sync_copy(dst_hbm.at[pl.ds(off, tail)], dst_t)
            pltpu.sync_copy(ones_t, acc.at[dst_t], add=True)
        plsc.subcore_barrier()
        pltpu.sync_copy(acc.at[my], stage)
        pltpu.sync_copy(stage, out_hbm.at[pl.ds((cid * NS + sid) * rpt, rpt)])

    f = pl.kernel(
        body,
        out_type=jax.ShapeDtypeStruct((NC * n_pad,), jnp.float32),
        mesh=_sc_mesh(),
        scratch_types=[
            pltpu.VMEM_SHARED((n_pad,), jnp.float32),
            pltpu.VMEM((rpt,), jnp.float32),
            pltpu.VMEM((CH,), jnp.int32),
            pltpu.VMEM((CH,), jnp.float32),
            pltpu.VMEM((max(tail, 8),), jnp.int32),
            pltpu.VMEM((max(tail, 8),), jnp.float32),
        ],
    )
    return f(dst, zeros, ones), n_pad


def _scatter_add_call(g, src, dst, n_nodes):
    E = src.shape[0]
    d = g.shape[1]
    e_sc, e_w = E // NC, E // NW
    n_full, tail = divmod(e_w, CH)
    nstage = 4
    rpt = ((n_nodes + NS - 1) // NS + 8 * nstage - 1) // (8 * nstage) * (8 * nstage)
    spt = rpt // nstage
    n_pad = NS * rpt
    zeros = jnp.zeros((spt, d), jnp.float32)

    def body(g_hbm, src_hbm, dst_hbm, zeros_hbm, out_hbm,
             acc, stage, src_v, dst_v, buf, src_t, dst_t, buf_t, sem):
        cid = lax.axis_index("c")
        sid = lax.axis_index("s")
        pltpu.sync_copy(zeros_hbm, stage)
        for k in range(nstage):
            pltpu.sync_copy(stage, acc.at[pl.ds(sid * rpt + k * spt, spt)])
        plsc.subcore_barrier()
        base = cid * e_sc + sid * e_w

        def chunk(i):
            off = pl.multiple_of(base + i * CH, 8)
            pltpu.sync_copy(src_hbm.at[pl.ds(off, CH)], src_v)
            pltpu.sync_copy(dst_hbm.at[pl.ds(off, CH)], dst_v)
            pltpu.async_copy(g_hbm.at[src_v], buf, sem).wait()
            pltpu.sync_copy(buf, acc.at[dst_v], add=True)

        lax.fori_loop(0, n_full, lambda i, c: (chunk(i), c)[1], 0)
        if tail:
            off = pl.multiple_of(base + n_full * CH, 8)
            pltpu.sync_copy(src_hbm.at[pl.ds(off, tail)], src_t)
            pltpu.sync_copy(dst_hbm.at[pl.ds(off, tail)], dst_t)
            pltpu.async_copy(g_hbm.at[src_t], buf_t, sem).wait()
            pltpu.sync_copy(buf_t, acc.at[dst_t], add=True)
        plsc.subcore_barrier()
        for k in range(nstage):
            pltpu.sync_copy(acc.at[pl.ds(sid * rpt + k * spt, spt)], stage)
            pltpu.sync_copy(
                stage,
                out_hbm.at[pl.ds((cid * NS + sid) * rpt + k * spt, spt)])

    f = pl.kernel(
        body,
        out_type=jax.ShapeDtypeStruct((NC * n_pad, d), jnp.float32),
        mesh=_sc_mesh(),
        scratch_types=[
            pltpu.VMEM_SHARED((n_pad, d), jnp.float32),
            pltpu.VMEM((spt, d), jnp.float32),
            pltpu.VMEM((CH,), jnp.int32),
            pltpu.VMEM((CH,), jnp.int32),
            pltpu.VMEM((CH, d), jnp.float32),
            pltpu.VMEM((max(tail, 8),), jnp.int32),
            pltpu.VMEM((max(tail, 8),), jnp.int32),
            pltpu.VMEM((max(tail, 8), d), jnp.float32),
            pltpu.SemaphoreType.DMA,
        ],
        compiler_params=pltpu.CompilerParams(use_tc_tiling_on_sc=(d % 128 == 0)),
    )
    return f(g, src, dst, zeros), n_pad


def _pre1(d2_ref, x_ref, w1_ref, g_ref, dis_ref):
    dsum = d2_ref[:, 0:1] + d2_ref[:, 1:2]
    pos = dsum > 0
    dis = jnp.where(pos, lax.rsqrt(jnp.where(pos, dsum, 1.0)), 0.0)
    h = jnp.dot(x_ref[...], w1_ref[...], preferred_element_type=jnp.float32)
    g_ref[...] = h * dis
    dis_ref[...] = dis


def _mid(a0_ref, a1_ref, dis_ref, b1_ref, w2_ref, g2_ref):
    dis = dis_ref[...]
    z = jnp.maximum((a0_ref[...] + a1_ref[...]) * dis + b1_ref[...], 0.0)
    h2 = jnp.dot(z, w2_ref[...], preferred_element_type=jnp.float32)
    g2_ref[...] = h2 * dis


def _fin(a0_ref, a1_ref, dis_ref, b2_ref, out_ref):
    y = (a0_ref[...] + a1_ref[...]) * dis_ref[...] + b2_ref[...]
    m = jnp.max(y, axis=1, keepdims=True)
    e = jnp.exp(y - m)
    s = jnp.sum(e, axis=1, keepdims=True)
    out_ref[...] = y - m - jnp.log(s)


def kernel(x, edge_index, W1, b1, W2, b2):
    n, f_in = x.shape
    hid = W1.shape[1]
    out_dim = W2.shape[1]
    dpad = 8
    src = edge_index[0]
    dst = edge_index[1]

    degs, n_pad1 = _deg_call(dst, n)
    d2 = degs.reshape(NC, n_pad1)[:, :n].T

    g1, dis = pl.pallas_call(
        _pre1,
        out_shape=(jax.ShapeDtypeStruct((n, hid), jnp.float32),
                   jax.ShapeDtypeStruct((n, 1), jnp.float32)),
    )(d2, x, W1)

    acc1, n_pad = _scatter_add_call(g1, src, dst, n)
    a = acc1.reshape(NC, n_pad, hid)

    w2p = jnp.concatenate(
        [W2, jnp.zeros((hid, dpad - out_dim), jnp.float32)], axis=1)
    g2 = pl.pallas_call(
        _mid,
        out_shape=jax.ShapeDtypeStruct((n, dpad), jnp.float32),
    )(a[0, :n], a[1, :n], dis, b1.reshape(1, hid), w2p)

    acc2, n_pad2 = _scatter_add_call(g2, src, dst, n)
    a2 = acc2.reshape(NC, n_pad2, dpad)

    b2p = jnp.concatenate(
        [b2, jnp.full((dpad - out_dim,), -1e30, jnp.float32)]).reshape(1, dpad)
    out8 = pl.pallas_call(
        _fin,
        out_shape=jax.ShapeDtypeStruct((n, dpad), jnp.float32),
    )(a2[0, :n], a2[1, :n], dis, b2p)

    out = out8[:, :out_dim]
    return (out, out, out, out)
